# baseline (speedup 1.0000x reference)
import functools

import jax
import jax.numpy as jnp
from jax import lax
from jax.experimental import pallas as pl
from jax.experimental.pallas import tpu as pltpu
from jax.experimental.pallas import tpu_sc as plsc

B, N, D = 64, 576, 768
HID, CD, CS = 512, 256, 512
TOK = B * N
BT = 1536
NT = TOK // BT

NW = 32
CHUNK = 64
CPW = TOK // (NW * CHUNK)


def _enc_vq_body(x_ref, w1_ref, b1_ref, w2_ref, b2_ref, cb_ref, dcb_ref,
                 recon_ref, idx_ref, commit_ref):
    i = pl.program_id(0)
    h = jnp.maximum(
        jnp.dot(x_ref[...], w1_ref[...], preferred_element_type=jnp.float32)
        + b1_ref[...], 0.0)
    z = jnp.dot(h, w2_ref[...], preferred_element_type=jnp.float32) + b2_ref[...]
    cb = cb_ref[...]
    cbm2 = -2.0 * cb
    s = lax.dot_general(z, cbm2, (((1,), (1,)), ((), ())),
                        preferred_element_type=jnp.float32)
    c2 = jnp.sum(cb * cb, axis=1)
    dist = s + c2[None, :]
    minval = jnp.min(dist, axis=1, keepdims=True)
    ids = lax.broadcasted_iota(jnp.int32, dist.shape, 1)
    idx = jnp.min(jnp.where(dist == minval, ids, CS), axis=1)
    idx_ref[0, 0, :] = idx
    onehot = jnp.where(ids == idx[:, None], 1.0, 0.0)
    recon_ref[...] = jnp.dot(onehot, dcb_ref[...],
                             preferred_element_type=jnp.float32)
    part = jnp.sum(minval) + jnp.sum(z * z)
    prev = jnp.where(i == 0, 0.0, commit_ref[...][0, 0])
    tot = prev + part
    out = jnp.where(i == NT - 1, tot / float(TOK * CD), tot)
    commit_ref[...] = jnp.broadcast_to(out, (1, 1))


def _dec_cb_body(cb_ref, wd1_ref, bd1_ref, wd2_ref, bd2_ref, dcb_ref):
    hd = jnp.maximum(
        jnp.dot(cb_ref[...], wd1_ref[...], preferred_element_type=jnp.float32)
        + bd1_ref[...], 0.0)
    dcb_ref[...] = jnp.dot(hd, wd2_ref[...],
                           preferred_element_type=jnp.float32) + bd2_ref[...]


def _gather_body(dcb_hbm, idx_hbm, out_hbm, idx_v, rows0, rows1,
                 gsem0, gsem1, osem0, osem1):
    wid = lax.axis_index("s") * 2 + lax.axis_index("c")
    base = wid * CPW * CHUNK
    pltpu.sync_copy(idx_hbm.at[pl.ds(base, CPW * CHUNK)], idx_v)
    bufs = (rows0, rows1)
    gsems = (gsem0, gsem1)
    osems = (osem0, osem1)
    out_h = [None] * CPW
    gat_h = [None] * CPW
    gat_h[0] = pltpu.async_copy(
        dcb_hbm.at[idx_v.at[pl.ds(0, CHUNK)]], bufs[0], gsems[0])
    for j in range(CPW):
        b = j % 2
        if j + 1 < CPW:
            nb = (j + 1) % 2
            if j >= 1:
                out_h[j - 1].wait()
            gat_h[j + 1] = pltpu.async_copy(
                dcb_hbm.at[idx_v.at[pl.ds((j + 1) * CHUNK, CHUNK)]],
                bufs[nb], gsems[nb])
        gat_h[j].wait()
        out_h[j] = pltpu.async_copy(
            bufs[b], out_hbm.at[pl.ds(base + j * CHUNK, CHUNK)], osems[b])
    out_h[CPW - 2].wait()
    out_h[CPW - 1].wait()


def _enc_vq(x2, w1, b1, w2, b2, cb, dcb):
    return pl.pallas_call(
        _enc_vq_body,
        grid=(NT,),
        in_specs=[
            pl.BlockSpec((BT, D), lambda i: (i, 0)),
            pl.BlockSpec((D, HID), lambda i: (0, 0)),
            pl.BlockSpec((1, HID), lambda i: (0, 0)),
            pl.BlockSpec((HID, CD), lambda i: (0, 0)),
            pl.BlockSpec((1, CD), lambda i: (0, 0)),
            pl.BlockSpec((CS, CD), lambda i: (0, 0)),
            pl.BlockSpec((CS, D), lambda i: (0, 0)),
        ],
        out_specs=[
            pl.BlockSpec((BT, D), lambda i: (i, 0)),
            pl.BlockSpec((1, 1, BT), lambda i: (i, 0, 0)),
            pl.BlockSpec((1, 1), lambda i: (0, 0)),
        ],
        out_shape=[
            jax.ShapeDtypeStruct((TOK, D), jnp.float32),
            jax.ShapeDtypeStruct((NT, 1, BT), jnp.int32),
            jax.ShapeDtypeStruct((1, 1), jnp.float32),
        ],
    )(x2, w1, b1, w2, b2, cb, dcb)


def _dec_cb(cb, wd1, bd1, wd2, bd2):
    return pl.pallas_call(
        _dec_cb_body,
        out_shape=jax.ShapeDtypeStruct((CS, D), jnp.float32),
    )(cb, wd1, bd1, wd2, bd2)


@functools.cache
def _make_gather():
    return functools.partial(
        pl.kernel,
        mesh=plsc.VectorSubcoreMesh(core_axis_name="c", subcore_axis_name="s"),
        out_type=jax.ShapeDtypeStruct((TOK, D), jnp.float32),
        scratch_types=[
            pltpu.VMEM((CPW * CHUNK,), jnp.int32),
            pltpu.VMEM((CHUNK, D), jnp.float32),
            pltpu.VMEM((CHUNK, D), jnp.float32),
            pltpu.SemaphoreType.DMA,
            pltpu.SemaphoreType.DMA,
            pltpu.SemaphoreType.DMA,
            pltpu.SemaphoreType.DMA,
        ],
    )(_gather_body)


def kernel(x, W1, b1, W2, b2, codebook, Wd1, bd1, Wd2, bd2):
    x2 = x.reshape(TOK, D)
    dcb = _dec_cb(codebook, Wd1, bd1.reshape(1, HID), Wd2, bd2.reshape(1, D))
    recon, idx_blk, commit = _enc_vq(x2, W1, b1.reshape(1, HID), W2,
                                     b2.reshape(1, CD), codebook, dcb)
    idx_flat = idx_blk.reshape(TOK)
    return (recon.reshape(B, N, D), idx_flat.reshape(B, N),
            commit.reshape(()))


# speedup vs baseline: 1.0117x; 1.0117x over previous
import jax
import jax.numpy as jnp
from jax import lax
from jax.experimental import pallas as pl

B, N, D = 64, 576, 768
HID, CD, CS = 512, 256, 512
TOK = B * N
BT = 2304
NT = TOK // BT


def _enc_vq_body(x_ref, w1_ref, b1_ref, w2_ref, b2_ref, cb_ref, dcb_ref,
                 recon_ref, idx_ref, commit_ref):
    i = pl.program_id(0)
    h = jnp.maximum(
        jnp.dot(x_ref[...], w1_ref[...], preferred_element_type=jnp.float32)
        + b1_ref[...], 0.0)
    z = jnp.dot(h, w2_ref[...], preferred_element_type=jnp.float32) + b2_ref[...]
    cb = cb_ref[...]
    cbm2 = -2.0 * cb
    s = lax.dot_general(z, cbm2, (((1,), (1,)), ((), ())),
                        preferred_element_type=jnp.float32)
    c2 = jnp.sum(cb * cb, axis=1)
    dist = s + c2[None, :]
    minval = jnp.min(dist, axis=1, keepdims=True)
    ids = lax.broadcasted_iota(jnp.int32, dist.shape, 1)
    idx = jnp.min(jnp.where(dist == minval, ids, CS), axis=1)
    idx_ref[0, 0, :] = idx
    onehot = jnp.where(ids == idx[:, None], 1.0, 0.0)
    recon_ref[...] = jnp.dot(onehot, dcb_ref[...],
                             preferred_element_type=jnp.float32)
    part = jnp.sum(minval) + jnp.sum(z * z)
    prev = jnp.where(i == 0, 0.0, commit_ref[...][0, 0])
    tot = prev + part
    out = jnp.where(i == NT - 1, tot / float(TOK * CD), tot)
    commit_ref[...] = jnp.broadcast_to(out, (1, 1))


def _dec_cb_body(cb_ref, wd1_ref, bd1_ref, wd2_ref, bd2_ref, dcb_ref):
    hd = jnp.maximum(
        jnp.dot(cb_ref[...], wd1_ref[...], preferred_element_type=jnp.float32)
        + bd1_ref[...], 0.0)
    dcb_ref[...] = jnp.dot(hd, wd2_ref[...],
                           preferred_element_type=jnp.float32) + bd2_ref[...]


def _enc_vq(x2, w1, b1, w2, b2, cb, dcb):
    return pl.pallas_call(
        _enc_vq_body,
        grid=(NT,),
        in_specs=[
            pl.BlockSpec((BT, D), lambda i: (i, 0)),
            pl.BlockSpec((D, HID), lambda i: (0, 0)),
            pl.BlockSpec((1, HID), lambda i: (0, 0)),
            pl.BlockSpec((HID, CD), lambda i: (0, 0)),
            pl.BlockSpec((1, CD), lambda i: (0, 0)),
            pl.BlockSpec((CS, CD), lambda i: (0, 0)),
            pl.BlockSpec((CS, D), lambda i: (0, 0)),
        ],
        out_specs=[
            pl.BlockSpec((BT, D), lambda i: (i, 0)),
            pl.BlockSpec((1, 1, BT), lambda i: (i, 0, 0)),
            pl.BlockSpec((1, 1), lambda i: (0, 0)),
        ],
        out_shape=[
            jax.ShapeDtypeStruct((TOK, D), jnp.float32),
            jax.ShapeDtypeStruct((NT, 1, BT), jnp.int32),
            jax.ShapeDtypeStruct((1, 1), jnp.float32),
        ],
    )(x2, w1, b1, w2, b2, cb, dcb)


def _dec_cb(cb, wd1, bd1, wd2, bd2):
    return pl.pallas_call(
        _dec_cb_body,
        out_shape=jax.ShapeDtypeStruct((CS, D), jnp.float32),
    )(cb, wd1, bd1, wd2, bd2)


def kernel(x, W1, b1, W2, b2, codebook, Wd1, bd1, Wd2, bd2):
    x2 = x.reshape(TOK, D)
    dcb = _dec_cb(codebook, Wd1, bd1.reshape(1, HID), Wd2, bd2.reshape(1, D))
    recon, idx_blk, commit = _enc_vq(x2, W1, b1.reshape(1, HID), W2,
                                     b2.reshape(1, CD), codebook, dcb)
    idx_flat = idx_blk.reshape(TOK)
    return (recon.reshape(B, N, D), idx_flat.reshape(B, N),
            commit.reshape(()))


# speedup vs baseline: 1.0376x; 1.0256x over previous
import jax
import jax.numpy as jnp
from jax import lax
from jax.experimental import pallas as pl

B, N, D = 64, 576, 768
HID, CD, CS = 512, 256, 512
TOK = B * N
BT = 2304
NT = TOK // BT


def _enc_vq_body(x_ref, w1_ref, b1_ref, w2_ref, b2_ref, cb_ref, dcb_ref,
                 revi_ref, recon_ref, idx_ref, commit_ref):
    i = pl.program_id(0)
    h = jnp.maximum(
        jnp.dot(x_ref[...], w1_ref[...], preferred_element_type=jnp.float32)
        + b1_ref[...], 0.0)
    z = jnp.dot(h, w2_ref[...], preferred_element_type=jnp.float32) + b2_ref[...]
    cb = cb_ref[...]
    cbm2 = -2.0 * cb
    s = lax.dot_general(z, cbm2, (((1,), (1,)), ((), ())),
                        preferred_element_type=jnp.float32)
    c2 = jnp.sum(cb * cb, axis=1)
    dist = s + c2[None, :]
    minval = jnp.min(dist, axis=1, keepdims=True)
    wrev = jnp.where(dist == minval, revi_ref[...], -1.0)
    rowmax = jnp.max(wrev, axis=1, keepdims=True)
    idx_ref[0, 0, :] = (jnp.float32(CS - 1) - rowmax[:, 0]).astype(jnp.int32)
    onehot = jnp.where(wrev == rowmax, 1.0, 0.0)
    recon_ref[...] = jnp.dot(onehot, dcb_ref[...],
                             preferred_element_type=jnp.float32)
    part = jnp.sum(minval) + jnp.sum(z * z)
    prev = jnp.where(i == 0, 0.0, commit_ref[...][0, 0])
    tot = prev + part
    out = jnp.where(i == NT - 1, tot / float(TOK * CD), tot)
    commit_ref[...] = jnp.broadcast_to(out, (1, 1))


def _dec_cb_body(cb_ref, wd1_ref, bd1_ref, wd2_ref, bd2_ref, dcb_ref):
    hd = jnp.maximum(
        jnp.dot(cb_ref[...], wd1_ref[...], preferred_element_type=jnp.float32)
        + bd1_ref[...], 0.0)
    dcb_ref[...] = jnp.dot(hd, wd2_ref[...],
                           preferred_element_type=jnp.float32) + bd2_ref[...]


def _enc_vq(x2, w1, b1, w2, b2, cb, dcb, revi):
    return pl.pallas_call(
        _enc_vq_body,
        grid=(NT,),
        in_specs=[
            pl.BlockSpec((BT, D), lambda i: (i, 0)),
            pl.BlockSpec((D, HID), lambda i: (0, 0)),
            pl.BlockSpec((1, HID), lambda i: (0, 0)),
            pl.BlockSpec((HID, CD), lambda i: (0, 0)),
            pl.BlockSpec((1, CD), lambda i: (0, 0)),
            pl.BlockSpec((CS, CD), lambda i: (0, 0)),
            pl.BlockSpec((CS, D), lambda i: (0, 0)),
            pl.BlockSpec((1, CS), lambda i: (0, 0)),
        ],
        out_specs=[
            pl.BlockSpec((BT, D), lambda i: (i, 0)),
            pl.BlockSpec((1, 1, BT), lambda i: (i, 0, 0)),
            pl.BlockSpec((1, 1), lambda i: (0, 0)),
        ],
        out_shape=[
            jax.ShapeDtypeStruct((TOK, D), jnp.float32),
            jax.ShapeDtypeStruct((NT, 1, BT), jnp.int32),
            jax.ShapeDtypeStruct((1, 1), jnp.float32),
        ],
    )(x2, w1, b1, w2, b2, cb, dcb, revi)


def _dec_cb(cb, wd1, bd1, wd2, bd2):
    return pl.pallas_call(
        _dec_cb_body,
        out_shape=jax.ShapeDtypeStruct((CS, D), jnp.float32),
    )(cb, wd1, bd1, wd2, bd2)


def kernel(x, W1, b1, W2, b2, codebook, Wd1, bd1, Wd2, bd2):
    x2 = x.reshape(TOK, D)
    dcb = _dec_cb(codebook, Wd1, bd1.reshape(1, HID), Wd2, bd2.reshape(1, D))
    revi = jnp.arange(CS - 1, -1, -1, dtype=jnp.float32).reshape(1, CS)
    recon, idx_blk, commit = _enc_vq(x2, W1, b1.reshape(1, HID), W2,
                                     b2.reshape(1, CD), codebook, dcb, revi)
    idx_flat = idx_blk.reshape(TOK)
    return (recon.reshape(B, N, D), idx_flat.reshape(B, N),
            commit.reshape(()))


# speedup vs baseline: 1.0714x; 1.0327x over previous
import jax
import jax.numpy as jnp
from jax import lax
from jax.experimental import pallas as pl

B, N, D = 64, 576, 768
HID, CD, CS = 512, 256, 512
TOK = B * N
BT = 2304
NT = TOK // BT


def _enc_vq_body(x_ref, w1_ref, b1_ref, w2_ref, b2_ref, cb_ref, dcb_ref,
                 revi_ref, recon_ref, idx_ref, commit_ref):
    i = pl.program_id(0)
    h = jnp.maximum(
        jnp.dot(x_ref[...], w1_ref[...], preferred_element_type=jnp.float32)
        + b1_ref[...], 0.0)
    z = jnp.dot(h, w2_ref[...], preferred_element_type=jnp.float32) + b2_ref[...]
    cb = cb_ref[...]
    cbm2 = -2.0 * cb
    s = lax.dot_general(z, cbm2, (((1,), (1,)), ((), ())),
                        preferred_element_type=jnp.float32)
    c2 = jnp.sum(cb * cb, axis=1)
    dist = s + c2[None, :]
    minval = jnp.min(dist, axis=1, keepdims=True)
    wrev = jnp.where(dist == minval, revi_ref[...], -1.0)
    rowmax = jnp.max(wrev, axis=1, keepdims=True)
    idx_ref[...] = jnp.float32(CS - 1) - rowmax
    onehot = jnp.where(wrev == rowmax, 1.0, 0.0)
    recon_ref[...] = jnp.dot(onehot, dcb_ref[...],
                             preferred_element_type=jnp.float32)
    part = jnp.sum(minval) + jnp.sum(z * z)
    prev = jnp.where(i == 0, 0.0, commit_ref[...][0, 0])
    tot = prev + part
    out = jnp.where(i == NT - 1, tot / float(TOK * CD), tot)
    commit_ref[...] = jnp.broadcast_to(out, (1, 1))


def _dec_cb_body(cb_ref, wd1_ref, bd1_ref, wd2_ref, bd2_ref, dcb_ref):
    hd = jnp.maximum(
        jnp.dot(cb_ref[...], wd1_ref[...], preferred_element_type=jnp.float32)
        + bd1_ref[...], 0.0)
    dcb_ref[...] = jnp.dot(hd, wd2_ref[...],
                           preferred_element_type=jnp.float32) + bd2_ref[...]


def _enc_vq(x2, w1, b1, w2, b2, cb, dcb, revi):
    return pl.pallas_call(
        _enc_vq_body,
        grid=(NT,),
        in_specs=[
            pl.BlockSpec((BT, D), lambda i: (i, 0)),
            pl.BlockSpec((D, HID), lambda i: (0, 0)),
            pl.BlockSpec((1, HID), lambda i: (0, 0)),
            pl.BlockSpec((HID, CD), lambda i: (0, 0)),
            pl.BlockSpec((1, CD), lambda i: (0, 0)),
            pl.BlockSpec((CS, CD), lambda i: (0, 0)),
            pl.BlockSpec((CS, D), lambda i: (0, 0)),
            pl.BlockSpec((1, CS), lambda i: (0, 0)),
        ],
        out_specs=[
            pl.BlockSpec((BT, D), lambda i: (i, 0)),
            pl.BlockSpec((BT, 1), lambda i: (i, 0)),
            pl.BlockSpec((1, 1), lambda i: (0, 0)),
        ],
        out_shape=[
            jax.ShapeDtypeStruct((TOK, D), jnp.float32),
            jax.ShapeDtypeStruct((TOK, 1), jnp.float32),
            jax.ShapeDtypeStruct((1, 1), jnp.float32),
        ],
    )(x2, w1, b1, w2, b2, cb, dcb, revi)


def _dec_cb(cb, wd1, bd1, wd2, bd2):
    return pl.pallas_call(
        _dec_cb_body,
        out_shape=jax.ShapeDtypeStruct((CS, D), jnp.float32),
    )(cb, wd1, bd1, wd2, bd2)


def kernel(x, W1, b1, W2, b2, codebook, Wd1, bd1, Wd2, bd2):
    x2 = x.reshape(TOK, D)
    dcb = _dec_cb(codebook, Wd1, bd1.reshape(1, HID), Wd2, bd2.reshape(1, D))
    revi = jnp.arange(CS - 1, -1, -1, dtype=jnp.float32).reshape(1, CS)
    recon, idx_blk, commit = _enc_vq(x2, W1, b1.reshape(1, HID), W2,
                                     b2.reshape(1, CD), codebook, dcb, revi)
    idx = idx_blk.reshape(B, N).astype(jnp.int32)
    return (recon.reshape(B, N, D), idx, commit.reshape(()))


# speedup vs baseline: 1.0804x; 1.0084x over previous
import jax
import jax.numpy as jnp
from jax import lax
from jax.experimental import pallas as pl

B, N, D = 64, 576, 768
HID, CD, CS = 512, 256, 512
TOK = B * N
BT = 3072
NT = TOK // BT


def _enc_vq_body(x_ref, w1_ref, b1_ref, w2_ref, b2_ref, cb_ref, dcb_ref,
                 revi_ref, recon_ref, idx_ref, commit_ref):
    i = pl.program_id(0)
    h = jnp.maximum(
        jnp.dot(x_ref[...], w1_ref[...], preferred_element_type=jnp.float32)
        + b1_ref[...], 0.0)
    z = jnp.dot(h, w2_ref[...], preferred_element_type=jnp.float32) + b2_ref[...]
    cb = cb_ref[...]
    cbm2 = -2.0 * cb
    s = lax.dot_general(z, cbm2, (((1,), (1,)), ((), ())),
                        preferred_element_type=jnp.float32)
    c2 = jnp.sum(cb * cb, axis=1)
    dist = s + c2[None, :]
    minval = jnp.min(dist, axis=1, keepdims=True)
    wrev = jnp.where(dist == minval, revi_ref[...], -1.0)
    rowmax = jnp.max(wrev, axis=1, keepdims=True)
    idx_ref[...] = jnp.float32(CS - 1) - rowmax
    onehot = jnp.where(wrev == rowmax, 1.0, 0.0)
    recon_ref[...] = jnp.dot(onehot, dcb_ref[...],
                             preferred_element_type=jnp.float32)
    part = jnp.sum(minval) + jnp.sum(z * z)
    prev = jnp.where(i == 0, 0.0, commit_ref[...][0, 0])
    tot = prev + part
    out = jnp.where(i == NT - 1, tot / float(TOK * CD), tot)
    commit_ref[...] = jnp.broadcast_to(out, (1, 1))


def _dec_cb_body(cb_ref, wd1_ref, bd1_ref, wd2_ref, bd2_ref, dcb_ref):
    hd = jnp.maximum(
        jnp.dot(cb_ref[...], wd1_ref[...], preferred_element_type=jnp.float32)
        + bd1_ref[...], 0.0)
    dcb_ref[...] = jnp.dot(hd, wd2_ref[...],
                           preferred_element_type=jnp.float32) + bd2_ref[...]


def _enc_vq(x2, w1, b1, w2, b2, cb, dcb, revi):
    return pl.pallas_call(
        _enc_vq_body,
        grid=(NT,),
        in_specs=[
            pl.BlockSpec((BT, D), lambda i: (i, 0)),
            pl.BlockSpec((D, HID), lambda i: (0, 0)),
            pl.BlockSpec((1, HID), lambda i: (0, 0)),
            pl.BlockSpec((HID, CD), lambda i: (0, 0)),
            pl.BlockSpec((1, CD), lambda i: (0, 0)),
            pl.BlockSpec((CS, CD), lambda i: (0, 0)),
            pl.BlockSpec((CS, D), lambda i: (0, 0)),
            pl.BlockSpec((1, CS), lambda i: (0, 0)),
        ],
        out_specs=[
            pl.BlockSpec((BT, D), lambda i: (i, 0)),
            pl.BlockSpec((BT, 1), lambda i: (i, 0)),
            pl.BlockSpec((1, 1), lambda i: (0, 0)),
        ],
        out_shape=[
            jax.ShapeDtypeStruct((TOK, D), jnp.float32),
            jax.ShapeDtypeStruct((TOK, 1), jnp.float32),
            jax.ShapeDtypeStruct((1, 1), jnp.float32),
        ],
    )(x2, w1, b1, w2, b2, cb, dcb, revi)


def _dec_cb(cb, wd1, bd1, wd2, bd2):
    return pl.pallas_call(
        _dec_cb_body,
        out_shape=jax.ShapeDtypeStruct((CS, D), jnp.float32),
    )(cb, wd1, bd1, wd2, bd2)


def kernel(x, W1, b1, W2, b2, codebook, Wd1, bd1, Wd2, bd2):
    x2 = x.reshape(TOK, D)
    dcb = _dec_cb(codebook, Wd1, bd1.reshape(1, HID), Wd2, bd2.reshape(1, D))
    revi = jnp.arange(CS - 1, -1, -1, dtype=jnp.float32).reshape(1, CS)
    recon, idx_blk, commit = _enc_vq(x2, W1, b1.reshape(1, HID), W2,
                                     b2.reshape(1, CD), codebook, dcb, revi)
    idx = idx_blk.reshape(B, N).astype(jnp.int32)
    return (recon.reshape(B, N, D), idx, commit.reshape(()))


# speedup vs baseline: 1.0963x; 1.0147x over previous
import jax
import jax.numpy as jnp
from jax import lax
from jax.experimental import pallas as pl
from jax.experimental.pallas import tpu as pltpu

B, N, D = 64, 576, 768
HID, CD, CS = 512, 256, 512
TOK = B * N
BT = 3072
NT = TOK // BT


def _enc_vq_body(x_ref, w1_ref, b1_ref, w2_ref, b2_ref, cb_ref,
                 wd1_ref, bd1_ref, wd2_ref, bd2_ref, revi_ref,
                 recon_ref, idx_ref, commit_ref, dcb_ref):
    i = pl.program_id(0)

    @pl.when(i == 0)
    def _decode_codebook():
        hd = jnp.maximum(
            jnp.dot(cb_ref[...], wd1_ref[...],
                    preferred_element_type=jnp.float32) + bd1_ref[...], 0.0)
        dcb_ref[...] = jnp.dot(hd, wd2_ref[...],
                               preferred_element_type=jnp.float32) + bd2_ref[...]
    h = jnp.maximum(
        jnp.dot(x_ref[...], w1_ref[...], preferred_element_type=jnp.float32)
        + b1_ref[...], 0.0)
    z = jnp.dot(h, w2_ref[...], preferred_element_type=jnp.float32) + b2_ref[...]
    cb = cb_ref[...]
    cbm2 = -2.0 * cb
    s = lax.dot_general(z, cbm2, (((1,), (1,)), ((), ())),
                        preferred_element_type=jnp.float32)
    c2 = jnp.sum(cb * cb, axis=1)
    dist = s + c2[None, :]
    minval = jnp.min(dist, axis=1, keepdims=True)
    wrev = jnp.where(dist == minval, revi_ref[...], -1.0)
    rowmax = jnp.max(wrev, axis=1, keepdims=True)
    idx_ref[...] = jnp.float32(CS - 1) - rowmax
    onehot = jnp.where(wrev == rowmax, 1.0, 0.0)
    recon_ref[...] = jnp.dot(onehot, dcb_ref[...],
                             preferred_element_type=jnp.float32)
    part = jnp.sum(minval) + jnp.sum(z * z)
    prev = jnp.where(i == 0, 0.0, commit_ref[...][0, 0])
    tot = prev + part
    out = jnp.where(i == NT - 1, tot / float(TOK * CD), tot)
    commit_ref[...] = jnp.broadcast_to(out, (1, 1))


def _enc_vq(x2, w1, b1, w2, b2, cb, wd1, bd1, wd2, bd2, revi):
    return pl.pallas_call(
        _enc_vq_body,
        grid=(NT,),
        in_specs=[
            pl.BlockSpec((BT, D), lambda i: (i, 0)),
            pl.BlockSpec((D, HID), lambda i: (0, 0)),
            pl.BlockSpec((1, HID), lambda i: (0, 0)),
            pl.BlockSpec((HID, CD), lambda i: (0, 0)),
            pl.BlockSpec((1, CD), lambda i: (0, 0)),
            pl.BlockSpec((CS, CD), lambda i: (0, 0)),
            pl.BlockSpec((CD, HID), lambda i: (0, 0)),
            pl.BlockSpec((1, HID), lambda i: (0, 0)),
            pl.BlockSpec((HID, D), lambda i: (0, 0)),
            pl.BlockSpec((1, D), lambda i: (0, 0)),
            pl.BlockSpec((1, CS), lambda i: (0, 0)),
        ],
        out_specs=[
            pl.BlockSpec((BT, D), lambda i: (i, 0)),
            pl.BlockSpec((BT, 1), lambda i: (i, 0)),
            pl.BlockSpec((1, 1), lambda i: (0, 0)),
        ],
        out_shape=[
            jax.ShapeDtypeStruct((TOK, D), jnp.float32),
            jax.ShapeDtypeStruct((TOK, 1), jnp.float32),
            jax.ShapeDtypeStruct((1, 1), jnp.float32),
        ],
        scratch_shapes=[pltpu.VMEM((CS, D), jnp.float32)],
    )(x2, w1, b1, w2, b2, cb, wd1, bd1, wd2, bd2, revi)


def kernel(x, W1, b1, W2, b2, codebook, Wd1, bd1, Wd2, bd2):
    x2 = x.reshape(TOK, D)
    revi = jnp.arange(CS - 1, -1, -1, dtype=jnp.float32).reshape(1, CS)
    recon, idx_blk, commit = _enc_vq(x2, W1, b1.reshape(1, HID), W2,
                                     b2.reshape(1, CD), codebook,
                                     Wd1, bd1.reshape(1, HID), Wd2,
                                     bd2.reshape(1, D), revi)
    idx = idx_blk.reshape(B, N).astype(jnp.int32)
    return (recon.reshape(B, N, D), idx, commit.reshape(()))
